# Initial kernel scaffold; baseline (speedup 1.0000x reference)
import jax
import jax.numpy as jnp
from jax.experimental import pallas as pl


def kernel(hidden_states, ln_scale, ln_bias, gate_w, gate_proj_w, up_proj_w, down_proj_w):
    raise NotImplementedError("write your pallas kernel here")


# speedup vs baseline: 1.5163x; 1.5163x over previous
import jax
import jax.numpy as jnp
from jax.experimental import pallas as pl
from jax.experimental.pallas import tpu as pltpu

EPS = 1e-05
LN_EPS = 1e-05
CAPACITY_FACTOR = 1.0


def kernel(hidden_states, ln_scale, ln_bias, gate_w, gate_proj_w, up_proj_w, down_proj_w):
    B, S, H = hidden_states.shape
    hid = hidden_states.reshape(-1, H)
    N = hid.shape[0]
    E = gate_w.shape[0]
    I = gate_proj_w.shape[1]
    C = int(N * CAPACITY_FACTOR / E)
    C = max(C, 1)
    C = min(C, N)

    def _router_kernel(x_ref, gw_ref, scale_ref, bias_ref,
                       idx_ref, prob_ref, cnt_ref, aux_ref):
        x = x_ref[...]
        mean = jnp.mean(x, axis=1, keepdims=True)
        xc = x - mean
        var = jnp.mean(xc * xc, axis=1, keepdims=True)
        xn = xc * jax.lax.rsqrt(var + LN_EPS) * scale_ref[...] + bias_ref[...]
        logits = jax.lax.dot_general(
            xn, gw_ref[...], (((1,), (1,)), ((), ())),
            preferred_element_type=jnp.float32)

        colmax = jnp.max(logits, axis=0, keepdims=True)
        ex = jnp.exp(logits - colmax)
        denom = jnp.sum(ex, axis=0, keepdims=True)
        pfull = ex / denom

        rowmax = jnp.max(logits, axis=1, keepdims=True)
        lse = jnp.log(jnp.sum(jnp.exp(logits - rowmax), axis=1,
                              keepdims=True)) + rowmax
        aux_ref[...] = jnp.full((1, 1), 0.001, jnp.float32) * jnp.mean(lse * lse)

        iota_r = jax.lax.broadcasted_iota(jnp.int32, (N, E), 0)
        work = logits
        sel = jnp.zeros((N, E), dtype=jnp.bool_)
        idx_rows = []
        val_rows = []
        for _ in range(C):
            cur = jnp.max(work, axis=0, keepdims=True)
            cand = jnp.where(work == cur, iota_r, jnp.int32(N))
            amin = jnp.min(cand, axis=0, keepdims=True)
            hit = iota_r == amin
            sel = jnp.logical_or(sel, hit)
            work = jnp.where(hit, -jnp.inf, work)
            idx_rows.append(amin)
            val_rows.append(cur)
        top_idx = jnp.concatenate(idx_rows, axis=0)
        top_val = jnp.concatenate(val_rows, axis=0)
        idx_ref[...] = top_idx
        prob_ref[...] = jnp.exp(top_val - colmax) / denom
        cnt_ref[...] = jnp.sum(jnp.where(sel, pfull, 0.0), axis=1, keepdims=True)

    idx, prob, cnt, aux = pl.pallas_call(
        _router_kernel,
        out_shape=[
            jax.ShapeDtypeStruct((C, E), jnp.int32),
            jax.ShapeDtypeStruct((C, E), jnp.float32),
            jax.ShapeDtypeStruct((N, 1), jnp.float32),
            jax.ShapeDtypeStruct((1, 1), jnp.float32),
        ],
    )(hid, gate_w, ln_scale.reshape(1, H), ln_bias.reshape(1, H))

    def _ffn_kernel(idx_ref, prob_ref, hid_ref, cnt_ref, gp_ref, up_ref, dp_ref,
                    out_ref, xs_ref):
        e = pl.program_id(0)

        @pl.when(e == 0)
        def _():
            out_ref[...] = jnp.zeros_like(out_ref)

        for c in range(C):
            xs_ref[c, :] = hid_ref[idx_ref[c, e], :]
        x = xs_ref[...]
        g = jax.lax.dot_general(x, gp_ref[0], (((1,), (1,)), ((), ())),
                                preferred_element_type=jnp.float32)
        u = jax.lax.dot_general(x, up_ref[0], (((1,), (1,)), ((), ())),
                                preferred_element_type=jnp.float32)
        h = g * jax.nn.sigmoid(g) * u
        o = jax.lax.dot_general(h, dp_ref[0], (((1,), (1,)), ((), ())),
                                preferred_element_type=jnp.float32)
        for c in range(C):
            t = idx_ref[c, e]
            out_ref[t, :] = out_ref[t, :] + o[c, :] * prob_ref[c, e]

        @pl.when(e == pl.num_programs(0) - 1)
        def _():
            out_ref[...] = out_ref[...] / jnp.maximum(cnt_ref[...], EPS)

    out = pl.pallas_call(
        _ffn_kernel,
        grid=(E,),
        in_specs=[
            pl.BlockSpec(memory_space=pltpu.SMEM),
            pl.BlockSpec(memory_space=pltpu.SMEM),
            pl.BlockSpec((N, H), lambda e: (0, 0)),
            pl.BlockSpec((N, 1), lambda e: (0, 0)),
            pl.BlockSpec((1, I, H), lambda e: (e, 0, 0)),
            pl.BlockSpec((1, I, H), lambda e: (e, 0, 0)),
            pl.BlockSpec((1, H, I), lambda e: (e, 0, 0)),
        ],
        out_specs=pl.BlockSpec((N, H), lambda e: (0, 0)),
        out_shape=jax.ShapeDtypeStruct((N, H), jnp.float32),
        scratch_shapes=[pltpu.VMEM((C, H), jnp.float32)],
        compiler_params=pltpu.CompilerParams(
            dimension_semantics=("arbitrary",)),
    )(idx, prob, hid, cnt, gate_proj_w, up_proj_w, down_proj_w)

    return out.reshape(B, S, H), aux.reshape(())


# speedup vs baseline: 1.5434x; 1.0178x over previous
import jax
import jax.numpy as jnp
from jax.experimental import pallas as pl
from jax.experimental.pallas import tpu as pltpu

EPS = 1e-05
LN_EPS = 1e-05
CAPACITY_FACTOR = 1.0


def kernel(hidden_states, ln_scale, ln_bias, gate_w, gate_proj_w, up_proj_w, down_proj_w):
    B, S, H = hidden_states.shape
    hid = hidden_states.reshape(-1, H)
    N = hid.shape[0]
    E = gate_w.shape[0]
    I = gate_proj_w.shape[1]
    C = int(N * CAPACITY_FACTOR / E)
    C = max(C, 1)
    C = min(C, N)

    def _router_kernel(x_ref, gw_ref, scale_ref, bias_ref,
                       idx_ref, prob_ref, aux_ref):
        x = x_ref[...]
        mean = jnp.mean(x, axis=1, keepdims=True)
        xc = x - mean
        var = jnp.mean(xc * xc, axis=1, keepdims=True)
        xn = xc * jax.lax.rsqrt(var + LN_EPS) * scale_ref[...] + bias_ref[...]
        logits = jax.lax.dot_general(
            xn, gw_ref[...], (((1,), (1,)), ((), ())),
            preferred_element_type=jnp.float32)

        colmax = jnp.max(logits, axis=0, keepdims=True)
        ex = jnp.exp(logits - colmax)
        denom = jnp.sum(ex, axis=0, keepdims=True)
        pfull = ex / denom

        rowmax = jnp.max(logits, axis=1, keepdims=True)
        lse = jnp.log(jnp.sum(jnp.exp(logits - rowmax), axis=1,
                              keepdims=True)) + rowmax
        aux_ref[...] = jnp.full((1, 1), 0.001, jnp.float32) * jnp.mean(lse * lse)

        iota_r = jax.lax.broadcasted_iota(jnp.int32, (N, E), 0)
        work = pfull
        idx_rows = []
        val_rows = []
        for _ in range(C):
            cur = jnp.max(work, axis=0, keepdims=True)
            cand = jnp.where(work == cur, iota_r, jnp.int32(N))
            amin = jnp.min(cand, axis=0, keepdims=True)
            hit = iota_r == amin
            work = jnp.where(hit, -1.0, work)
            idx_rows.append(amin)
            val_rows.append(cur)
        idx_ref[...] = jnp.concatenate(idx_rows, axis=0)
        prob_ref[...] = jnp.concatenate(val_rows, axis=0)

    idx, prob, aux = pl.pallas_call(
        _router_kernel,
        out_shape=[
            jax.ShapeDtypeStruct((C, E), jnp.int32),
            jax.ShapeDtypeStruct((C, E), jnp.float32),
            jax.ShapeDtypeStruct((1, 1), jnp.float32),
        ],
    )(hid, gate_w, ln_scale.reshape(1, H), ln_bias.reshape(1, H))

    def _ffn_kernel(idx_ref, prob_ref, hid_ref, gp_ref, up_ref, dp_ref,
                    out_ref, xs_ref, cnt_ref):
        e = pl.program_id(0)

        @pl.when(e == 0)
        def _():
            out_ref[...] = jnp.zeros_like(out_ref)
            cnt_ref[...] = jnp.zeros_like(cnt_ref)

        for c in range(C):
            xs_ref[c, :] = hid_ref[idx_ref[c, e], :]
        x = xs_ref[...]
        g = jax.lax.dot_general(x, gp_ref[0], (((1,), (1,)), ((), ())),
                                preferred_element_type=jnp.float32)
        u = jax.lax.dot_general(x, up_ref[0], (((1,), (1,)), ((), ())),
                                preferred_element_type=jnp.float32)
        h = g * jax.nn.sigmoid(g) * u
        o = jax.lax.dot_general(h, dp_ref[0], (((1,), (1,)), ((), ())),
                                preferred_element_type=jnp.float32)
        for c in range(C):
            t = idx_ref[c, e]
            p = prob_ref[c, e]
            out_ref[t, :] = out_ref[t, :] + o[c, :] * p
            cnt_ref[pl.ds(t, 1), :] = cnt_ref[pl.ds(t, 1), :] + p

        @pl.when(e == pl.num_programs(0) - 1)
        def _():
            out_ref[...] = out_ref[...] / jnp.maximum(cnt_ref[...], EPS)

    out = pl.pallas_call(
        _ffn_kernel,
        grid=(E,),
        in_specs=[
            pl.BlockSpec(memory_space=pltpu.SMEM),
            pl.BlockSpec(memory_space=pltpu.SMEM),
            pl.BlockSpec((N, H), lambda e: (0, 0)),
            pl.BlockSpec((1, I, H), lambda e: (e, 0, 0)),
            pl.BlockSpec((1, I, H), lambda e: (e, 0, 0)),
            pl.BlockSpec((1, H, I), lambda e: (e, 0, 0)),
        ],
        out_specs=pl.BlockSpec((N, H), lambda e: (0, 0)),
        out_shape=jax.ShapeDtypeStruct((N, H), jnp.float32),
        scratch_shapes=[pltpu.VMEM((C, H), jnp.float32),
                        pltpu.VMEM((N, 1), jnp.float32)],
        compiler_params=pltpu.CompilerParams(
            dimension_semantics=("arbitrary",)),
    )(idx, prob, hid, gate_proj_w, up_proj_w, down_proj_w)

    return out.reshape(B, S, H), aux.reshape(())
